# baseline (speedup 1.0000x reference)
import functools

import jax
import jax.numpy as jnp
from jax import lax
from jax.experimental import pallas as pl
from jax.experimental.pallas import tpu as pltpu
from jax.experimental.pallas import tpu_sc as plsc

VOCAB = 100000
EMBED_DIM = 128
BATCH = 16384

_CHUNK = 128


def _make_gather(num_workers: int, b_per_w: int, n_chunks: int):
    mesh = plsc.VectorSubcoreMesh(core_axis_name="c", subcore_axis_name="s")

    @functools.partial(
        pl.kernel,
        mesh=mesh,
        out_type=jax.ShapeDtypeStruct(
            (num_workers, n_chunks, _CHUNK, EMBED_DIM), jnp.float32
        ),
        scratch_types=[
            pltpu.VMEM((n_chunks, _CHUNK), jnp.int32),
            pltpu.VMEM((n_chunks, _CHUNK, EMBED_DIM), jnp.float32),
        ]
        + [pltpu.SemaphoreType.DMA for _ in range(n_chunks + 1)]
        + [pltpu.SemaphoreType.DMA, pltpu.SemaphoreType.DMA, pltpu.SemaphoreType.DMA],
    )
    def gather_kernel(table_hbm, idx_hbm, out_hbm, idx_v, rows_v, *sems):
        gsems, wsem = sems[: n_chunks + 1], sems[n_chunks + 1]
        isem, isem2 = sems[n_chunks + 2], sems[n_chunks + 3]
        nc = lax.axis_index("c")
        sid = lax.axis_index("s")
        wid = sid * 2 + nc
        i0 = pltpu.async_copy(idx_hbm.at[wid, 0], idx_v.at[0], isem)
        irest = pltpu.async_copy(
            idx_hbm.at[wid, pl.ds(1, n_chunks - 1)],
            idx_v.at[pl.ds(1, n_chunks - 1)],
            isem2,
        )
        half = _CHUNK // 2
        chunks = [(j, 0, _CHUNK) for j in range(n_chunks - 1)]
        chunks += [(n_chunks - 1, 0, half), (n_chunks - 1, half, half)]

        def _gather(c, sem):
            j, off, ln = c
            return pltpu.async_copy(
                table_hbm.at[idx_v.at[j, pl.ds(off, ln)]],
                rows_v.at[j, pl.ds(off, ln)],
                sem,
            )

        i0.wait()
        gathers = [_gather(chunks[0], gsems[0])]
        irest.wait()
        gathers += [_gather(c, s) for c, s in zip(chunks[1:], gsems[1:])]
        writes = []
        for g, (j, off, ln) in zip(gathers, chunks):
            g.wait()
            writes.append(
                pltpu.async_copy(
                    rows_v.at[j, pl.ds(off, ln)],
                    out_hbm.at[wid, j, pl.ds(off, ln)],
                    wsem,
                )
            )
        for c in writes:
            c.wait()

    return gather_kernel


def kernel(data, in_table, out_table):
    del out_table
    info = plsc.get_sparse_core_info()
    num_workers = info.num_cores * info.num_subcores
    b_per_w = BATCH // num_workers
    n_chunks = b_per_w // _CHUNK
    idx = data.astype(jnp.int32).reshape(num_workers, n_chunks, _CHUNK)
    out = _make_gather(num_workers, b_per_w, n_chunks)(in_table, idx)
    return out.reshape(BATCH, EMBED_DIM)


# speedup vs baseline: 1.0090x; 1.0090x over previous
import functools

import jax
import jax.numpy as jnp
from jax import lax
from jax.experimental import pallas as pl
from jax.experimental.pallas import tpu as pltpu
from jax.experimental.pallas import tpu_sc as plsc

VOCAB = 100000
EMBED_DIM = 128
BATCH = 16384

_CHUNK = 128


def _make_gather(num_workers: int, n_chunks: int):
    mesh = plsc.VectorSubcoreMesh(core_axis_name="c", subcore_axis_name="s")

    @functools.partial(
        pl.kernel,
        mesh=mesh,
        out_type=jax.ShapeDtypeStruct(
            (num_workers, n_chunks, _CHUNK, EMBED_DIM), jnp.float32
        ),
        scratch_types=[
            pltpu.VMEM((n_chunks, _CHUNK), jnp.int32),
            pltpu.VMEM((n_chunks, _CHUNK, EMBED_DIM), jnp.float32),
            pltpu.SemaphoreType.DMA,
        ],
    )
    def gather_kernel(table_hbm, idx_hbm, out_hbm, idx_v, rows_v, sem):
        nc = lax.axis_index("c")
        sid = lax.axis_index("s")
        wid = sid * 2 + nc
        pltpu.sync_copy(idx_hbm.at[wid], idx_v)
        copies = [
            pltpu.async_copy(table_hbm.at[idx_v.at[j]], rows_v.at[j], sem)
            for j in range(n_chunks)
        ]
        for c in copies:
            c.wait()
        pltpu.sync_copy(rows_v, out_hbm.at[wid])

    return gather_kernel


def kernel(data, in_table, out_table):
    del out_table
    info = plsc.get_sparse_core_info()
    num_workers = info.num_cores * info.num_subcores
    n_chunks = BATCH // num_workers // _CHUNK
    idx = data.astype(jnp.int32).reshape(num_workers, n_chunks, _CHUNK)
    out = _make_gather(num_workers, n_chunks)(in_table, idx)
    return out.reshape(BATCH, EMBED_DIM)
